# baseline (speedup 1.0000x reference)
import functools

import jax
import jax.numpy as jnp
from jax import lax
from jax.experimental import pallas as pl
from jax.experimental.pallas import tpu as pltpu
from jax.experimental.pallas import tpu_sc as plsc

B, S, H = 2, 2048, 1024
I = 2048
E = 8
K = 2
EPS = 1e-06
T = B * S
R = T * K
BT = 512
NBR = T // BT
BLK = 128
BLK_SHIFT = 7
NPAD = R + E * BLK
NBLK = NPAD // BLK


def _router_body(x_ref, gw_ref, eid_ref, w_ref, aux_ref, acc_ref):
    n = pl.program_id(0)
    x = x_ref[...]
    gw = gw_ref[...]
    logits = jnp.dot(x, gw.T, preferred_element_type=jnp.float32)
    logits = jnp.clip(logits, -50.0, 50.0)
    m = jnp.max(logits, axis=-1, keepdims=True)
    ex = jnp.exp(logits - m)
    s = jnp.sum(ex, axis=-1, keepdims=True)
    probs = ex / s
    lse = m + jnp.log(s)
    zpart = jnp.sum(lse * lse)

    iota = jax.lax.broadcasted_iota(jnp.int32, (BT, E), 1)
    m1 = jnp.max(probs, axis=-1, keepdims=True)
    i1 = jnp.min(jnp.where(probs == m1, iota, E), axis=-1, keepdims=True)
    probs_m = jnp.where(iota == i1, -1.0, probs)
    m2 = jnp.max(probs_m, axis=-1, keepdims=True)
    i2 = jnp.min(jnp.where(probs_m == m2, iota, E), axis=-1, keepdims=True)
    denom = m1 + m2 + EPS
    eid_ref[...] = jnp.concatenate([i1, i2], axis=1)
    w_ref[...] = jnp.concatenate([m1 / denom, m2 / denom], axis=1)

    cnt = jnp.sum((iota == i1).astype(jnp.float32)
                  + (iota == i2).astype(jnp.float32), axis=0, keepdims=True)
    psum = jnp.sum(probs, axis=0, keepdims=True)
    ps_safe = jnp.clip(probs, EPS, 1.0)
    entpart = jnp.sum(-ps_safe * jnp.log(ps_safe))

    @pl.when(n == 0)
    def _():
        acc_ref[...] = jnp.zeros_like(acc_ref)

    acc_ref[0:1, 0:E] = acc_ref[0:1, 0:E] + cnt
    acc_ref[1:2, 0:E] = acc_ref[1:2, 0:E] + psum
    acc_ref[2:3, 0:1] = acc_ref[2:3, 0:1] + zpart
    acc_ref[3:4, 0:1] = acc_ref[3:4, 0:1] + entpart

    @pl.when(n == NBR - 1)
    def _():
        cnts = acc_ref[0:1, 0:E]
        ps = acc_ref[1:2, 0:E]
        lb = E * jnp.sum((cnts / R) * (ps / T))
        z = jnp.sum(acc_ref[2:3, 0:1]) / T * 0.001
        ent = (jnp.log(jnp.float32(E)) - jnp.sum(acc_ref[3:4, 0:1]) / T) * 0.01
        aux_ref[...] = jnp.broadcast_to(lb + z + ent, (1, 128))


def _router(x2d, gate_w):
    return pl.pallas_call(
        _router_body,
        grid=(NBR,),
        in_specs=[
            pl.BlockSpec((BT, H), lambda n: (n, 0)),
            pl.BlockSpec((E, H), lambda n: (0, 0)),
        ],
        out_specs=[
            pl.BlockSpec((BT, 2), lambda n: (n, 0)),
            pl.BlockSpec((BT, 2), lambda n: (n, 0)),
            pl.BlockSpec((1, 128), lambda n: (0, 0)),
        ],
        out_shape=[
            jax.ShapeDtypeStruct((T, 2), jnp.int32),
            jax.ShapeDtypeStruct((T, 2), jnp.float32),
            jax.ShapeDtypeStruct((1, 128), jnp.float32),
        ],
        scratch_shapes=[pltpu.VMEM((8, 128), jnp.float32)],
    )(x2d, gate_w)


def _moe_ffn_body(bexp_ref, nused_ref, xs_ref, wg_ref, wu_ref, wd_ref,
                  ws_ref, ys_ref):
    n = pl.program_id(0)

    @pl.when(n < nused_ref[0])
    def _():
        x = xs_ref[...]
        g = jnp.dot(x, wg_ref[0].T, preferred_element_type=jnp.float32)
        u = jnp.dot(x, wu_ref[0].T, preferred_element_type=jnp.float32)
        h = jax.nn.silu(g) * u
        part = jnp.dot(h, wd_ref[0].T, preferred_element_type=jnp.float32)
        ys_ref[...] = part * ws_ref[...]


def _moe_ffn(bexp, nused, xs, egw, euw, edw, ws2d):
    grid_spec = pltpu.PrefetchScalarGridSpec(
        num_scalar_prefetch=2,
        grid=(NBLK,),
        in_specs=[
            pl.BlockSpec((BLK, H), lambda n, bexp, nu: (n, 0)),
            pl.BlockSpec((1, I, H), lambda n, bexp, nu: (bexp[n], 0, 0)),
            pl.BlockSpec((1, I, H), lambda n, bexp, nu: (bexp[n], 0, 0)),
            pl.BlockSpec((1, H, I), lambda n, bexp, nu: (bexp[n], 0, 0)),
            pl.BlockSpec((BLK, 1), lambda n, bexp, nu: (n, 0)),
        ],
        out_specs=pl.BlockSpec((BLK, H), lambda n, bexp, nu: (n, 0)),
    )
    return pl.pallas_call(
        _moe_ffn_body,
        grid_spec=grid_spec,
        out_shape=jax.ShapeDtypeStruct((NPAD, H), jnp.float32),
    )(bexp, nused, xs, egw, euw, edw, ws2d)


def _shared_body(sgp_ref, x_ref, wg_ref, wu_ref, wd_ref, o_ref):
    x = x_ref[...]
    g = jnp.dot(x, wg_ref[...].T, preferred_element_type=jnp.float32)
    u = jnp.dot(x, wu_ref[...].T, preferred_element_type=jnp.float32)
    h = jax.nn.silu(g) * u
    part = jnp.dot(h, wd_ref[...].T, preferred_element_type=jnp.float32)
    scale = jax.nn.sigmoid(sgp_ref[0])
    o_ref[...] = part * scale


def _shared_ffn(x2d, sgw, suw, sdw, sgp):
    grid_spec = pltpu.PrefetchScalarGridSpec(
        num_scalar_prefetch=1,
        grid=(T // BLK,),
        in_specs=[
            pl.BlockSpec((BLK, H), lambda n, sgp: (n, 0)),
            pl.BlockSpec((I, H), lambda n, sgp: (0, 0)),
            pl.BlockSpec((I, H), lambda n, sgp: (0, 0)),
            pl.BlockSpec((H, I), lambda n, sgp: (0, 0)),
        ],
        out_specs=pl.BlockSpec((BLK, H), lambda n, sgp: (n, 0)),
    )
    return pl.pallas_call(
        _shared_body,
        grid_spec=grid_spec,
        out_shape=jax.ShapeDtypeStruct((T, H), jnp.float32),
    )(sgp, x2d, sgw, suw, sdw)


NC, NS, L = 2, 16, 16
NW = NC * NS
TPT = T // NW
RPT = R // NW
NV = R // L
NBLK_PAD = 80

_sc_mesh = plsc.VectorSubcoreMesh(core_axis_name="c", subcore_axis_name="s")


def _dispatch_body(eids_hbm, wts_hbm, x_hbm,
                   pos_hbm, ws_hbm, bexp_hbm, nused_hbm, xs_hbm,
                   eids_v, pos_loc, w_loc, idx128, idx0, idx1, xrows,
                   bexp_s, nused_s, sem_a, sem_b):
    wid = lax.axis_index("s") * NC + lax.axis_index("c")
    base_r = wid * RPT
    base_t = wid * TPT
    my_v = wid * (RPT // L)

    pltpu.sync_copy(eids_hbm, eids_v)

    def count_body(i, carry):
        v = eids_v[pl.ds(i * L, L)]
        pre = jnp.where(i < my_v, 1, 0)
        out = []
        for e in range(E):
            m = jnp.where(v == e, 1, 0)
            out.append(carry[e] + m)
            out.append(carry[E + e] + m * pre)
        return tuple(out[0::2]) + tuple(out[1::2])

    zero = tuple(jnp.zeros((L,), jnp.int32) for _ in range(2 * E))
    acc = lax.fori_loop(0, NV, count_body, zero)
    cnt_all = [jnp.sum(acc[e]) for e in range(E)]
    cnt_pre = [jnp.sum(acc[E + e]) for e in range(E)]

    pad = [((cnt_all[e] + (BLK - 1)) >> BLK_SHIFT) << BLK_SHIFT
           for e in range(E)]
    start = []
    run = jnp.int32(0)
    for e in range(E):
        start.append(run)
        run = run + pad[e]
    base = [start[e] + cnt_pre[e] for e in range(E)]

    running = list(base)
    for j in range(RPT // L):
        v = eids_v[pl.ds(base_r + j * L, L)]
        posv = jnp.zeros((L,), jnp.int32)
        for e in range(E):
            m = v == e
            mi = jnp.where(m, 1, 0)
            c = jnp.cumsum(mi)
            posv = jnp.where(m, running[e] + c - 1, posv)
            running[e] = running[e] + jnp.sum(mi)
        pos_loc[pl.ds(j * L, L)] = posv

    pltpu.sync_copy(pos_loc, pos_hbm.at[pl.ds(base_r, RPT)])

    pltpu.sync_copy(wts_hbm.at[pl.ds(base_r, RPT)], w_loc)
    for c in range(2):
        for j in range(8):
            idx128[pl.ds(j * L, L)] = pos_loc[pl.ds(c * 128 + j * L, L)]
        pltpu.async_copy(w_loc.at[pl.ds(c * 128, 128)],
                         ws_hbm.at[idx128], sem_a).wait()

    lane = lax.iota(jnp.int32, L)
    for c in range(2):
        t0 = base_t + c * 64
        pltpu.sync_copy(x_hbm.at[pl.ds(t0, 64)], xrows)
        for j in range(4):
            idxs = c * 128 + j * 32 + lane * 2
            idx0[pl.ds(j * L, L)] = plsc.load_gather(pos_loc, [idxs])
            idx1[pl.ds(j * L, L)] = plsc.load_gather(pos_loc, [idxs + 1])
        cp0 = pltpu.async_copy(xrows, xs_hbm.at[idx0], sem_a)
        cp1 = pltpu.async_copy(xrows, xs_hbm.at[idx1], sem_b)
        cp0.wait()
        cp1.wait()

    endblk = [(start[e] + pad[e]) >> BLK_SHIFT for e in range(E)]
    for j in range(NBLK_PAD // L):
        bid = j * L + lane
        sacc = jnp.zeros((L,), jnp.int32)
        for e in range(E):
            sacc = sacc + jnp.where(bid >= endblk[e], 1, 0)
        bexp_s[pl.ds(j * L, L)] = jnp.minimum(sacc, E - 1)
    nused_s[...] = jnp.zeros((L,), jnp.int32) + endblk[E - 1]

    @pl.when(wid == 0)
    def _():
        pltpu.sync_copy(bexp_s, bexp_hbm)
        pltpu.sync_copy(nused_s, nused_hbm)


def _dispatch(eids_flat, wts_flat, x2d):
    return pl.kernel(
        _dispatch_body,
        out_type=[
            jax.ShapeDtypeStruct((R,), jnp.int32),
            jax.ShapeDtypeStruct((NPAD,), jnp.float32),
            jax.ShapeDtypeStruct((NBLK_PAD,), jnp.int32),
            jax.ShapeDtypeStruct((L,), jnp.int32),
            jax.ShapeDtypeStruct((NPAD, H), jnp.float32),
        ],
        mesh=_sc_mesh,
        compiler_params=pltpu.CompilerParams(needs_layout_passes=False),
        scratch_types=[
            pltpu.VMEM((R,), jnp.int32),
            pltpu.VMEM((RPT,), jnp.int32),
            pltpu.VMEM((RPT,), jnp.float32),
            pltpu.VMEM((128,), jnp.int32),
            pltpu.VMEM((64,), jnp.int32),
            pltpu.VMEM((64,), jnp.int32),
            pltpu.VMEM((64, H), jnp.float32),
            pltpu.VMEM((NBLK_PAD,), jnp.int32),
            pltpu.VMEM((L,), jnp.int32),
            pltpu.SemaphoreType.DMA,
            pltpu.SemaphoreType.DMA,
        ],
    )(eids_flat, wts_flat, x2d)


def _combine_body(ys_hbm, ysh_hbm, pos_hbm, out_hbm,
                  pos_v, idx0, idx1, r0, r1, sh, ov, sem0, sem1):
    wid = lax.axis_index("s") * NC + lax.axis_index("c")
    base_t = wid * TPT
    base_r = wid * RPT
    pltpu.sync_copy(pos_hbm.at[pl.ds(base_r, RPT)], pos_v)
    lane = lax.iota(jnp.int32, L)
    for c in range(TPT // L):
        t0 = base_t + c * L
        idxs = c * 32 + lane * 2
        idx0[...] = plsc.load_gather(pos_v, [idxs])
        idx1[...] = plsc.load_gather(pos_v, [idxs + 1])
        cp0 = pltpu.async_copy(ys_hbm.at[idx0], r0, sem0)
        cp1 = pltpu.async_copy(ys_hbm.at[idx1], r1, sem1)
        pltpu.sync_copy(ysh_hbm.at[pl.ds(t0, L)], sh)
        cp0.wait()
        cp1.wait()

        def add_body(col, _):
            o = col * L
            for t in range(L):
                ov[t, pl.ds(o, L)] = (r0[t, pl.ds(o, L)]
                                      + r1[t, pl.ds(o, L)]
                                      + sh[t, pl.ds(o, L)])
            return 0

        lax.fori_loop(0, H // L, add_body, 0)
        pltpu.sync_copy(ov, out_hbm.at[pl.ds(t0, L)])


def _combine(ys, ysh, pos):
    return pl.kernel(
        _combine_body,
        out_type=jax.ShapeDtypeStruct((T, H), jnp.float32),
        mesh=_sc_mesh,
        compiler_params=pltpu.CompilerParams(needs_layout_passes=False),
        scratch_types=[
            pltpu.VMEM((RPT,), jnp.int32),
            pltpu.VMEM((L,), jnp.int32),
            pltpu.VMEM((L,), jnp.int32),
            pltpu.VMEM((L, H), jnp.float32),
            pltpu.VMEM((L, H), jnp.float32),
            pltpu.VMEM((L, H), jnp.float32),
            pltpu.VMEM((L, H), jnp.float32),
            pltpu.SemaphoreType.DMA,
            pltpu.SemaphoreType.DMA,
        ],
    )(ys, ysh, pos)


def kernel(hidden_states, gate_w, expert_gate_w, expert_up_w, expert_down_w,
           shared_gate_w, shared_up_w, shared_down_w, shared_gate_param):
    x2d = hidden_states.reshape(T, H)

    eids, wts, aux = _router(x2d, gate_w)

    pos, ws, bexp, nused, xs = _dispatch(eids.reshape(R), wts.reshape(R), x2d)

    ys = _moe_ffn(bexp, nused, xs, expert_gate_w, expert_up_w, expert_down_w,
                  ws.reshape(NPAD, 1))
    ysh = _shared_ffn(x2d, shared_gate_w, shared_up_w, shared_down_w,
                      shared_gate_param)

    out = _combine(ys, ysh, pos)

    return out.reshape(B, S, H), aux[0, 0]


# speedup vs baseline: 1.6893x; 1.6893x over previous
import functools

import jax
import jax.numpy as jnp
from jax import lax
from jax.experimental import pallas as pl
from jax.experimental.pallas import tpu as pltpu
from jax.experimental.pallas import tpu_sc as plsc

B, S, H = 2, 2048, 1024
I = 2048
E = 8
K = 2
EPS = 1e-06
T = B * S
R = T * K
BT = 512
NBR = T // BT
BLK = 256
BLK_SHIFT = 8
NPAD = R + E * BLK
NBLK = NPAD // BLK


def _router_body(x_ref, gw_ref, eid_ref, w_ref, aux_ref, acc_ref):
    n = pl.program_id(0)
    x = x_ref[...]
    gw = gw_ref[...]
    logits = jnp.dot(x, gw.T, preferred_element_type=jnp.float32)
    logits = jnp.clip(logits, -50.0, 50.0)
    m = jnp.max(logits, axis=-1, keepdims=True)
    ex = jnp.exp(logits - m)
    s = jnp.sum(ex, axis=-1, keepdims=True)
    probs = ex / s
    lse = m + jnp.log(s)
    zpart = jnp.sum(lse * lse)

    iota = jax.lax.broadcasted_iota(jnp.int32, (BT, E), 1)
    m1 = jnp.max(probs, axis=-1, keepdims=True)
    i1 = jnp.min(jnp.where(probs == m1, iota, E), axis=-1, keepdims=True)
    probs_m = jnp.where(iota == i1, -1.0, probs)
    m2 = jnp.max(probs_m, axis=-1, keepdims=True)
    i2 = jnp.min(jnp.where(probs_m == m2, iota, E), axis=-1, keepdims=True)
    denom = m1 + m2 + EPS
    eid_ref[...] = jnp.concatenate([i1, i2], axis=1)
    w_ref[...] = jnp.concatenate([m1 / denom, m2 / denom], axis=1)

    cnt = jnp.sum((iota == i1).astype(jnp.float32)
                  + (iota == i2).astype(jnp.float32), axis=0, keepdims=True)
    psum = jnp.sum(probs, axis=0, keepdims=True)
    ps_safe = jnp.clip(probs, EPS, 1.0)
    entpart = jnp.sum(-ps_safe * jnp.log(ps_safe))

    @pl.when(n == 0)
    def _():
        acc_ref[...] = jnp.zeros_like(acc_ref)

    acc_ref[0:1, 0:E] = acc_ref[0:1, 0:E] + cnt
    acc_ref[1:2, 0:E] = acc_ref[1:2, 0:E] + psum
    acc_ref[2:3, 0:1] = acc_ref[2:3, 0:1] + zpart
    acc_ref[3:4, 0:1] = acc_ref[3:4, 0:1] + entpart

    @pl.when(n == NBR - 1)
    def _():
        cnts = acc_ref[0:1, 0:E]
        ps = acc_ref[1:2, 0:E]
        lb = E * jnp.sum((cnts / R) * (ps / T))
        z = jnp.sum(acc_ref[2:3, 0:1]) / T * 0.001
        ent = (jnp.log(jnp.float32(E)) - jnp.sum(acc_ref[3:4, 0:1]) / T) * 0.01
        aux_ref[...] = jnp.broadcast_to(lb + z + ent, (1, 128))


def _router(x2d, gate_w):
    return pl.pallas_call(
        _router_body,
        grid=(NBR,),
        in_specs=[
            pl.BlockSpec((BT, H), lambda n: (n, 0)),
            pl.BlockSpec((E, H), lambda n: (0, 0)),
        ],
        out_specs=[
            pl.BlockSpec((BT, 2), lambda n: (n, 0)),
            pl.BlockSpec((BT, 2), lambda n: (n, 0)),
            pl.BlockSpec((1, 128), lambda n: (0, 0)),
        ],
        out_shape=[
            jax.ShapeDtypeStruct((T, 2), jnp.int32),
            jax.ShapeDtypeStruct((T, 2), jnp.float32),
            jax.ShapeDtypeStruct((1, 128), jnp.float32),
        ],
        scratch_shapes=[pltpu.VMEM((8, 128), jnp.float32)],
    )(x2d, gate_w)


def _moe_ffn_body(bexp_ref, nused_ref, xs_ref, wg_ref, wu_ref, wd_ref,
                  ws_ref, ys_ref):
    n = pl.program_id(0)

    @pl.when(n < nused_ref[0])
    def _():
        x = xs_ref[...]
        g = jnp.dot(x, wg_ref[0].T, preferred_element_type=jnp.float32)
        u = jnp.dot(x, wu_ref[0].T, preferred_element_type=jnp.float32)
        h = jax.nn.silu(g) * u
        part = jnp.dot(h, wd_ref[0].T, preferred_element_type=jnp.float32)
        ys_ref[...] = part * ws_ref[...]


def _moe_ffn(bexp, nused, xs, egw, euw, edw, ws2d):
    grid_spec = pltpu.PrefetchScalarGridSpec(
        num_scalar_prefetch=2,
        grid=(NBLK,),
        in_specs=[
            pl.BlockSpec((BLK, H), lambda n, bexp, nu: (n, 0)),
            pl.BlockSpec((1, I, H), lambda n, bexp, nu: (bexp[n], 0, 0)),
            pl.BlockSpec((1, I, H), lambda n, bexp, nu: (bexp[n], 0, 0)),
            pl.BlockSpec((1, H, I), lambda n, bexp, nu: (bexp[n], 0, 0)),
            pl.BlockSpec((BLK, 1), lambda n, bexp, nu: (n, 0)),
        ],
        out_specs=pl.BlockSpec((BLK, H), lambda n, bexp, nu: (n, 0)),
    )
    return pl.pallas_call(
        _moe_ffn_body,
        grid_spec=grid_spec,
        out_shape=jax.ShapeDtypeStruct((NPAD, H), jnp.float32),
    )(bexp, nused, xs, egw, euw, edw, ws2d)


def _shared_body(sgp_ref, x_ref, wg_ref, wu_ref, wd_ref, o_ref):
    x = x_ref[...]
    g = jnp.dot(x, wg_ref[...].T, preferred_element_type=jnp.float32)
    u = jnp.dot(x, wu_ref[...].T, preferred_element_type=jnp.float32)
    h = jax.nn.silu(g) * u
    part = jnp.dot(h, wd_ref[...].T, preferred_element_type=jnp.float32)
    scale = jax.nn.sigmoid(sgp_ref[0])
    o_ref[...] = part * scale


def _shared_ffn(x2d, sgw, suw, sdw, sgp):
    grid_spec = pltpu.PrefetchScalarGridSpec(
        num_scalar_prefetch=1,
        grid=(T // BLK,),
        in_specs=[
            pl.BlockSpec((BLK, H), lambda n, sgp: (n, 0)),
            pl.BlockSpec((I, H), lambda n, sgp: (0, 0)),
            pl.BlockSpec((I, H), lambda n, sgp: (0, 0)),
            pl.BlockSpec((H, I), lambda n, sgp: (0, 0)),
        ],
        out_specs=pl.BlockSpec((BLK, H), lambda n, sgp: (n, 0)),
    )
    return pl.pallas_call(
        _shared_body,
        grid_spec=grid_spec,
        out_shape=jax.ShapeDtypeStruct((T, H), jnp.float32),
    )(sgp, x2d, sgw, suw, sdw)


NC, NS, L = 2, 16, 16
NW = NC * NS
TPT = T // NW
RPT = R // NW
NV = R // L
NBLK_PAD = 48

_sc_mesh = plsc.VectorSubcoreMesh(core_axis_name="c", subcore_axis_name="s")


def _dispatch_body(eids_hbm, wts_hbm, x_hbm,
                   pos_hbm, ws_hbm, bexp_hbm, nused_hbm, xs_hbm,
                   eids_v, pos_loc, w_loc, idx128, idx0, idx1, xrows,
                   bexp_s, nused_s, sem_a, sem_b):
    wid = lax.axis_index("s") * NC + lax.axis_index("c")
    base_r = wid * RPT
    base_t = wid * TPT
    my_v = wid * (RPT // L)

    pltpu.sync_copy(eids_hbm, eids_v)

    def count_body(i, carry):
        v = eids_v[pl.ds(i * L, L)]
        pre = jnp.where(i < my_v, 1, 0)
        out = []
        for e in range(E):
            m = jnp.where(v == e, 1, 0)
            out.append(carry[e] + m)
            out.append(carry[E + e] + m * pre)
        return tuple(out[0::2]) + tuple(out[1::2])

    zero = tuple(jnp.zeros((L,), jnp.int32) for _ in range(2 * E))
    acc = lax.fori_loop(0, NV, count_body, zero)
    cnt_all = [jnp.sum(acc[e]) for e in range(E)]
    cnt_pre = [jnp.sum(acc[E + e]) for e in range(E)]

    pad = [((cnt_all[e] + (BLK - 1)) >> BLK_SHIFT) << BLK_SHIFT
           for e in range(E)]
    start = []
    run = jnp.int32(0)
    for e in range(E):
        start.append(run)
        run = run + pad[e]
    base = [start[e] + cnt_pre[e] for e in range(E)]

    running = list(base)
    for j in range(RPT // L):
        v = eids_v[pl.ds(base_r + j * L, L)]
        posv = jnp.zeros((L,), jnp.int32)
        for e in range(E):
            m = v == e
            mi = jnp.where(m, 1, 0)
            c = jnp.cumsum(mi)
            posv = jnp.where(m, running[e] + c - 1, posv)
            running[e] = running[e] + jnp.sum(mi)
        pos_loc[pl.ds(j * L, L)] = posv

    pltpu.sync_copy(pos_loc, pos_hbm.at[pl.ds(base_r, RPT)])

    pltpu.sync_copy(wts_hbm.at[pl.ds(base_r, RPT)], w_loc)
    for c in range(2):
        for j in range(8):
            idx128[pl.ds(j * L, L)] = pos_loc[pl.ds(c * 128 + j * L, L)]
        pltpu.async_copy(w_loc.at[pl.ds(c * 128, 128)],
                         ws_hbm.at[idx128], sem_a).wait()

    lane = lax.iota(jnp.int32, L)
    for c in range(2):
        t0 = base_t + c * 64
        pltpu.sync_copy(x_hbm.at[pl.ds(t0, 64)], xrows)
        for j in range(4):
            idxs = c * 128 + j * 32 + lane * 2
            idx0[pl.ds(j * L, L)] = plsc.load_gather(pos_loc, [idxs])
            idx1[pl.ds(j * L, L)] = plsc.load_gather(pos_loc, [idxs + 1])
        cp0 = pltpu.async_copy(xrows, xs_hbm.at[idx0], sem_a)
        cp1 = pltpu.async_copy(xrows, xs_hbm.at[idx1], sem_b)
        cp0.wait()
        cp1.wait()

    endblk = [(start[e] + pad[e]) >> BLK_SHIFT for e in range(E)]
    for j in range(NBLK_PAD // L):
        bid = j * L + lane
        sacc = jnp.zeros((L,), jnp.int32)
        for e in range(E):
            sacc = sacc + jnp.where(bid >= endblk[e], 1, 0)
        bexp_s[pl.ds(j * L, L)] = jnp.minimum(sacc, E - 1)
    nused_s[...] = jnp.zeros((L,), jnp.int32) + endblk[E - 1]

    @pl.when(wid == 0)
    def _():
        pltpu.sync_copy(bexp_s, bexp_hbm)
        pltpu.sync_copy(nused_s, nused_hbm)


def _dispatch(eids_flat, wts_flat, x2d):
    return pl.kernel(
        _dispatch_body,
        out_type=[
            jax.ShapeDtypeStruct((R,), jnp.int32),
            jax.ShapeDtypeStruct((NPAD,), jnp.float32),
            jax.ShapeDtypeStruct((NBLK_PAD,), jnp.int32),
            jax.ShapeDtypeStruct((L,), jnp.int32),
            jax.ShapeDtypeStruct((NPAD, H), jnp.float32),
        ],
        mesh=_sc_mesh,
        compiler_params=pltpu.CompilerParams(needs_layout_passes=False),
        scratch_types=[
            pltpu.VMEM((R,), jnp.int32),
            pltpu.VMEM((RPT,), jnp.int32),
            pltpu.VMEM((RPT,), jnp.float32),
            pltpu.VMEM((128,), jnp.int32),
            pltpu.VMEM((64,), jnp.int32),
            pltpu.VMEM((64,), jnp.int32),
            pltpu.VMEM((64, H), jnp.float32),
            pltpu.VMEM((NBLK_PAD,), jnp.int32),
            pltpu.VMEM((L,), jnp.int32),
            pltpu.SemaphoreType.DMA,
            pltpu.SemaphoreType.DMA,
        ],
    )(eids_flat, wts_flat, x2d)


def _combine_body(ys_hbm, ysh_hbm, pos_hbm, out_hbm,
                  pos_v, idx0, idx1, r0, r1, sh, ov, sem0, sem1):
    wid = lax.axis_index("s") * NC + lax.axis_index("c")
    base_t = wid * TPT
    base_r = wid * RPT
    pltpu.sync_copy(pos_hbm.at[pl.ds(base_r, RPT)], pos_v)
    lane = lax.iota(jnp.int32, L)
    for c in range(TPT // L):
        t0 = base_t + c * L
        idxs = c * 32 + lane * 2
        idx0[...] = plsc.load_gather(pos_v, [idxs])
        idx1[...] = plsc.load_gather(pos_v, [idxs + 1])
        cp0 = pltpu.async_copy(ys_hbm.at[idx0], r0, sem0)
        cp1 = pltpu.async_copy(ys_hbm.at[idx1], r1, sem1)
        pltpu.sync_copy(ysh_hbm.at[pl.ds(t0, L)], sh)
        cp0.wait()
        cp1.wait()

        def add_body(col, _):
            o = col * L
            for t in range(L):
                ov[t, pl.ds(o, L)] = (r0[t, pl.ds(o, L)]
                                      + r1[t, pl.ds(o, L)]
                                      + sh[t, pl.ds(o, L)])
            return 0

        lax.fori_loop(0, H // L, add_body, 0)
        pltpu.sync_copy(ov, out_hbm.at[pl.ds(t0, L)])


def _combine(ys, ysh, pos):
    return pl.kernel(
        _combine_body,
        out_type=jax.ShapeDtypeStruct((T, H), jnp.float32),
        mesh=_sc_mesh,
        compiler_params=pltpu.CompilerParams(needs_layout_passes=False),
        scratch_types=[
            pltpu.VMEM((RPT,), jnp.int32),
            pltpu.VMEM((L,), jnp.int32),
            pltpu.VMEM((L,), jnp.int32),
            pltpu.VMEM((L, H), jnp.float32),
            pltpu.VMEM((L, H), jnp.float32),
            pltpu.VMEM((L, H), jnp.float32),
            pltpu.VMEM((L, H), jnp.float32),
            pltpu.SemaphoreType.DMA,
            pltpu.SemaphoreType.DMA,
        ],
    )(ys, ysh, pos)


def kernel(hidden_states, gate_w, expert_gate_w, expert_up_w, expert_down_w,
           shared_gate_w, shared_up_w, shared_down_w, shared_gate_param):
    x2d = hidden_states.reshape(T, H)

    eids, wts, aux = _router(x2d, gate_w)

    pos, ws, bexp, nused, xs = _dispatch(eids.reshape(R), wts.reshape(R), x2d)

    ys = _moe_ffn(bexp, nused, xs, expert_gate_w, expert_up_w, expert_down_w,
                  ws.reshape(NPAD, 1))
    ysh = _shared_ffn(x2d, shared_gate_w, shared_up_w, shared_down_w,
                      shared_gate_param)

    out = _combine(ys, ysh, pos)

    return out.reshape(B, S, H), aux[0, 0]


# speedup vs baseline: 1.7872x; 1.0580x over previous
import functools

import jax
import jax.numpy as jnp
from jax import lax
from jax.experimental import pallas as pl
from jax.experimental.pallas import tpu as pltpu
from jax.experimental.pallas import tpu_sc as plsc

B, S, H = 2, 2048, 1024
I = 2048
E = 8
K = 2
EPS = 1e-06
T = B * S
R = T * K
BT = 512
NBR = T // BT
BLK = 256
BLK_SHIFT = 8
NPAD = R + E * BLK
NBLK = NPAD // BLK


def _router_body(x_ref, gw_ref, eid_ref, w_ref, aux_ref, acc_ref):
    n = pl.program_id(0)
    x = x_ref[...]
    gw = gw_ref[...]
    logits = jnp.dot(x, gw.T, preferred_element_type=jnp.float32)
    logits = jnp.clip(logits, -50.0, 50.0)
    m = jnp.max(logits, axis=-1, keepdims=True)
    ex = jnp.exp(logits - m)
    s = jnp.sum(ex, axis=-1, keepdims=True)
    probs = ex / s
    lse = m + jnp.log(s)
    zpart = jnp.sum(lse * lse)

    iota = jax.lax.broadcasted_iota(jnp.int32, (BT, E), 1)
    m1 = jnp.max(probs, axis=-1, keepdims=True)
    i1 = jnp.min(jnp.where(probs == m1, iota, E), axis=-1, keepdims=True)
    probs_m = jnp.where(iota == i1, -1.0, probs)
    m2 = jnp.max(probs_m, axis=-1, keepdims=True)
    i2 = jnp.min(jnp.where(probs_m == m2, iota, E), axis=-1, keepdims=True)
    denom = m1 + m2 + EPS
    eid_ref[...] = jnp.concatenate([i1, i2], axis=1)
    w_ref[...] = jnp.concatenate([m1 / denom, m2 / denom], axis=1)

    cnt = jnp.sum((iota == i1).astype(jnp.float32)
                  + (iota == i2).astype(jnp.float32), axis=0, keepdims=True)
    psum = jnp.sum(probs, axis=0, keepdims=True)
    ps_safe = jnp.clip(probs, EPS, 1.0)
    entpart = jnp.sum(-ps_safe * jnp.log(ps_safe))

    @pl.when(n == 0)
    def _():
        acc_ref[...] = jnp.zeros_like(acc_ref)

    acc_ref[0:1, 0:E] = acc_ref[0:1, 0:E] + cnt
    acc_ref[1:2, 0:E] = acc_ref[1:2, 0:E] + psum
    acc_ref[2:3, 0:1] = acc_ref[2:3, 0:1] + zpart
    acc_ref[3:4, 0:1] = acc_ref[3:4, 0:1] + entpart

    @pl.when(n == NBR - 1)
    def _():
        cnts = acc_ref[0:1, 0:E]
        ps = acc_ref[1:2, 0:E]
        lb = E * jnp.sum((cnts / R) * (ps / T))
        z = jnp.sum(acc_ref[2:3, 0:1]) / T * 0.001
        ent = (jnp.log(jnp.float32(E)) - jnp.sum(acc_ref[3:4, 0:1]) / T) * 0.01
        aux_ref[...] = jnp.broadcast_to(lb + z + ent, (1, 128))


def _router(x2d, gate_w):
    return pl.pallas_call(
        _router_body,
        grid=(NBR,),
        in_specs=[
            pl.BlockSpec((BT, H), lambda n: (n, 0)),
            pl.BlockSpec((E, H), lambda n: (0, 0)),
        ],
        out_specs=[
            pl.BlockSpec((BT, 2), lambda n: (n, 0)),
            pl.BlockSpec((BT, 2), lambda n: (n, 0)),
            pl.BlockSpec((1, 128), lambda n: (0, 0)),
        ],
        out_shape=[
            jax.ShapeDtypeStruct((T, 2), jnp.int32),
            jax.ShapeDtypeStruct((T, 2), jnp.float32),
            jax.ShapeDtypeStruct((1, 128), jnp.float32),
        ],
        scratch_shapes=[pltpu.VMEM((8, 128), jnp.float32)],
    )(x2d, gate_w)


def _moe_ffn_body(bexp_ref, nused_ref, xs_ref, wg_ref, wu_ref, wd_ref,
                  ws_ref, ys_ref):
    n = pl.program_id(0)

    @pl.when(n < nused_ref[0])
    def _():
        x = xs_ref[...]
        g = jnp.dot(x, wg_ref[0].T, preferred_element_type=jnp.float32)
        u = jnp.dot(x, wu_ref[0].T, preferred_element_type=jnp.float32)
        h = jax.nn.silu(g) * u
        part = jnp.dot(h, wd_ref[0].T, preferred_element_type=jnp.float32)
        ys_ref[...] = part * ws_ref[...]


def _moe_ffn(bexp, nused, xs, egw, euw, edw, ws2d):
    grid_spec = pltpu.PrefetchScalarGridSpec(
        num_scalar_prefetch=2,
        grid=(NBLK,),
        in_specs=[
            pl.BlockSpec((BLK, H), lambda n, bexp, nu: (n, 0)),
            pl.BlockSpec((1, I, H), lambda n, bexp, nu: (bexp[n], 0, 0)),
            pl.BlockSpec((1, I, H), lambda n, bexp, nu: (bexp[n], 0, 0)),
            pl.BlockSpec((1, H, I), lambda n, bexp, nu: (bexp[n], 0, 0)),
            pl.BlockSpec((BLK, 1), lambda n, bexp, nu: (n, 0)),
        ],
        out_specs=pl.BlockSpec((BLK, H), lambda n, bexp, nu: (n, 0)),
    )
    return pl.pallas_call(
        _moe_ffn_body,
        grid_spec=grid_spec,
        out_shape=jax.ShapeDtypeStruct((NPAD, H), jnp.float32),
    )(bexp, nused, xs, egw, euw, edw, ws2d)


def _shared_body(sgp_ref, x_ref, wg_ref, wu_ref, wd_ref, o_ref):
    x = x_ref[...]
    g = jnp.dot(x, wg_ref[...].T, preferred_element_type=jnp.float32)
    u = jnp.dot(x, wu_ref[...].T, preferred_element_type=jnp.float32)
    h = jax.nn.silu(g) * u
    part = jnp.dot(h, wd_ref[...].T, preferred_element_type=jnp.float32)
    scale = jax.nn.sigmoid(sgp_ref[0])
    o_ref[...] = part * scale


def _shared_ffn(x2d, sgw, suw, sdw, sgp):
    grid_spec = pltpu.PrefetchScalarGridSpec(
        num_scalar_prefetch=1,
        grid=(T // BLK,),
        in_specs=[
            pl.BlockSpec((BLK, H), lambda n, sgp: (n, 0)),
            pl.BlockSpec((I, H), lambda n, sgp: (0, 0)),
            pl.BlockSpec((I, H), lambda n, sgp: (0, 0)),
            pl.BlockSpec((H, I), lambda n, sgp: (0, 0)),
        ],
        out_specs=pl.BlockSpec((BLK, H), lambda n, sgp: (n, 0)),
    )
    return pl.pallas_call(
        _shared_body,
        grid_spec=grid_spec,
        out_shape=jax.ShapeDtypeStruct((T, H), jnp.float32),
    )(sgp, x2d, sgw, suw, sdw)


NC, NS, L = 2, 16, 16
NW = NC * NS
TPT = T // NW
RPT = R // NW
NV = R // L
NBLK_PAD = 48

_sc_mesh = plsc.VectorSubcoreMesh(core_axis_name="c", subcore_axis_name="s")


def _dispatch_body(eids_hbm, wts_hbm, x_hbm,
                   pos_hbm, ws_hbm, bexp_hbm, nused_hbm, xs_hbm,
                   eids_v, pos_loc, w_loc, idx128, idx0, idx1, xrows,
                   bexp_s, nused_s, sem_a, sem_b):
    wid = lax.axis_index("s") * NC + lax.axis_index("c")
    base_r = wid * RPT
    base_t = wid * TPT
    my_v = wid * (RPT // L)

    pltpu.sync_copy(eids_hbm, eids_v)

    def count_body(i, carry):
        v = eids_v[pl.ds(i * L, L)]
        pre = jnp.where(i < my_v, 1, 0)
        out = []
        for e in range(E):
            m = jnp.where(v == e, 1, 0)
            out.append(carry[e] + m)
            out.append(carry[E + e] + m * pre)
        return tuple(out[0::2]) + tuple(out[1::2])

    zero = tuple(jnp.zeros((L,), jnp.int32) for _ in range(2 * E))
    acc = lax.fori_loop(0, NV, count_body, zero)
    cnt_all = [jnp.sum(acc[e]) for e in range(E)]
    cnt_pre = [jnp.sum(acc[E + e]) for e in range(E)]

    pad = [((cnt_all[e] + (BLK - 1)) >> BLK_SHIFT) << BLK_SHIFT
           for e in range(E)]
    start = []
    run = jnp.int32(0)
    for e in range(E):
        start.append(run)
        run = run + pad[e]
    base = [start[e] + cnt_pre[e] for e in range(E)]

    running = list(base)
    for j in range(RPT // L):
        v = eids_v[pl.ds(base_r + j * L, L)]
        posv = jnp.zeros((L,), jnp.int32)
        for e in range(E):
            m = v == e
            mi = jnp.where(m, 1, 0)
            c = jnp.cumsum(mi)
            posv = jnp.where(m, running[e] + c - 1, posv)
            running[e] = running[e] + jnp.sum(mi)
        pos_loc[pl.ds(j * L, L)] = posv

    pltpu.sync_copy(pos_loc, pos_hbm.at[pl.ds(base_r, RPT)])

    pltpu.sync_copy(wts_hbm.at[pl.ds(base_r, RPT)], w_loc)
    for c in range(2):
        for j in range(8):
            idx128[pl.ds(j * L, L)] = pos_loc[pl.ds(c * 128 + j * L, L)]
        pltpu.async_copy(w_loc.at[pl.ds(c * 128, 128)],
                         ws_hbm.at[idx128], sem_a).wait()

    lane = lax.iota(jnp.int32, L)
    for c in range(2):
        t0 = base_t + c * 64
        pltpu.sync_copy(x_hbm.at[pl.ds(t0, 64)], xrows)
        for j in range(4):
            idxs = c * 128 + j * 32 + lane * 2
            idx0[pl.ds(j * L, L)] = plsc.load_gather(pos_loc, [idxs])
            idx1[pl.ds(j * L, L)] = plsc.load_gather(pos_loc, [idxs + 1])
        cp0 = pltpu.async_copy(xrows, xs_hbm.at[idx0], sem_a)
        cp1 = pltpu.async_copy(xrows, xs_hbm.at[idx1], sem_b)
        cp0.wait()
        cp1.wait()

    endblk = [(start[e] + pad[e]) >> BLK_SHIFT for e in range(E)]
    for j in range(NBLK_PAD // L):
        bid = j * L + lane
        sacc = jnp.zeros((L,), jnp.int32)
        for e in range(E):
            sacc = sacc + jnp.where(bid >= endblk[e], 1, 0)
        bexp_s[pl.ds(j * L, L)] = jnp.minimum(sacc, E - 1)
    nused_s[...] = jnp.zeros((L,), jnp.int32) + endblk[E - 1]

    @pl.when(wid == 0)
    def _():
        pltpu.sync_copy(bexp_s, bexp_hbm)
        pltpu.sync_copy(nused_s, nused_hbm)


def _dispatch(eids_flat, wts_flat, x2d):
    return pl.kernel(
        _dispatch_body,
        out_type=[
            jax.ShapeDtypeStruct((R,), jnp.int32),
            jax.ShapeDtypeStruct((NPAD,), jnp.float32),
            jax.ShapeDtypeStruct((NBLK_PAD,), jnp.int32),
            jax.ShapeDtypeStruct((L,), jnp.int32),
            jax.ShapeDtypeStruct((NPAD, H), jnp.float32),
        ],
        mesh=_sc_mesh,
        compiler_params=pltpu.CompilerParams(needs_layout_passes=False),
        scratch_types=[
            pltpu.VMEM((R,), jnp.int32),
            pltpu.VMEM((RPT,), jnp.int32),
            pltpu.VMEM((RPT,), jnp.float32),
            pltpu.VMEM((128,), jnp.int32),
            pltpu.VMEM((64,), jnp.int32),
            pltpu.VMEM((64,), jnp.int32),
            pltpu.VMEM((64, H), jnp.float32),
            pltpu.VMEM((NBLK_PAD,), jnp.int32),
            pltpu.VMEM((L,), jnp.int32),
            pltpu.SemaphoreType.DMA,
            pltpu.SemaphoreType.DMA,
        ],
    )(eids_flat, wts_flat, x2d)


NCH = TPT // L


def _combine_body(ys_hbm, ysh_hbm, pos_hbm, out_hbm,
                  pos_v, idx0, idx1, r0, r1, sh, ov, gsems):
    wid = lax.axis_index("s") * NC + lax.axis_index("c")
    base_t = wid * TPT
    base_r = wid * RPT
    pltpu.sync_copy(pos_hbm.at[pl.ds(base_r, RPT)], pos_v)
    lane = lax.iota(jnp.int32, L)

    def issue(c, b):
        t0 = base_t + c * L
        idxs = c * 32 + lane * 2
        idx0[b, pl.ds(0, L)] = plsc.load_gather(pos_v, [idxs])
        idx1[b, pl.ds(0, L)] = plsc.load_gather(pos_v, [idxs + 1])
        pltpu.async_copy(ys_hbm.at[idx0.at[b]], r0.at[b], gsems.at[b])
        pltpu.async_copy(ys_hbm.at[idx1.at[b]], r1.at[b], gsems.at[b])
        pltpu.async_copy(ysh_hbm.at[pl.ds(t0, L)], sh.at[b], gsems.at[b])

    issue(0, 0)
    for c in range(NCH):
        b = c % 2
        if c + 1 < NCH:
            issue(c + 1, (c + 1) % 2)
        pltpu.make_async_copy(ys_hbm.at[pl.ds(0, L)], r0.at[b],
                              gsems.at[b]).wait()
        pltpu.make_async_copy(ys_hbm.at[pl.ds(0, L)], r1.at[b],
                              gsems.at[b]).wait()
        pltpu.make_async_copy(ysh_hbm.at[pl.ds(0, L)], sh.at[b],
                              gsems.at[b]).wait()
        def add_body(col, _):
            o = col * L
            for t in range(L):
                ov[t, pl.ds(o, L)] = (r0[b, t, pl.ds(o, L)]
                                      + r1[b, t, pl.ds(o, L)]
                                      + sh[b, t, pl.ds(o, L)])
            return 0

        lax.fori_loop(0, H // L, add_body, 0)
        pltpu.sync_copy(ov, out_hbm.at[pl.ds(base_t + c * L, L)])


def _combine(ys, ysh, pos):
    return pl.kernel(
        _combine_body,
        out_type=jax.ShapeDtypeStruct((T, H), jnp.float32),
        mesh=_sc_mesh,
        compiler_params=pltpu.CompilerParams(needs_layout_passes=False),
        scratch_types=[
            pltpu.VMEM((RPT,), jnp.int32),
            pltpu.VMEM((2, L), jnp.int32),
            pltpu.VMEM((2, L), jnp.int32),
            pltpu.VMEM((2, L, H), jnp.float32),
            pltpu.VMEM((2, L, H), jnp.float32),
            pltpu.VMEM((2, L, H), jnp.float32),
            pltpu.VMEM((L, H), jnp.float32),
            pltpu.SemaphoreType.DMA((2,)),
        ],
    )(ys, ysh, pos)


def kernel(hidden_states, gate_w, expert_gate_w, expert_up_w, expert_down_w,
           shared_gate_w, shared_up_w, shared_down_w, shared_gate_param):
    x2d = hidden_states.reshape(T, H)

    eids, wts, aux = _router(x2d, gate_w)

    pos, ws, bexp, nused, xs = _dispatch(eids.reshape(R), wts.reshape(R), x2d)

    ys = _moe_ffn(bexp, nused, xs, expert_gate_w, expert_up_w, expert_down_w,
                  ws.reshape(NPAD, 1))
    ysh = _shared_ffn(x2d, shared_gate_w, shared_up_w, shared_down_w,
                      shared_gate_param)

    out = _combine(ys, ysh, pos)

    return out.reshape(B, S, H), aux[0, 0]


# speedup vs baseline: 1.7923x; 1.0029x over previous
import functools

import jax
import jax.numpy as jnp
from jax import lax
from jax.experimental import pallas as pl
from jax.experimental.pallas import tpu as pltpu
from jax.experimental.pallas import tpu_sc as plsc

B, S, H = 2, 2048, 1024
I = 2048
E = 8
K = 2
EPS = 1e-06
T = B * S
R = T * K
BT = 512
NBR = T // BT
BLK = 256
BLK_SHIFT = 8
NPAD = R + E * BLK
NBLK = NPAD // BLK


def _router_body(x_ref, gw_ref, eid_ref, w_ref, aux_ref, cntb_ref, acc_ref):
    n = pl.program_id(0)
    x = x_ref[...]
    gw = gw_ref[...]
    logits = jnp.dot(x, gw.T, preferred_element_type=jnp.float32)
    logits = jnp.clip(logits, -50.0, 50.0)
    m = jnp.max(logits, axis=-1, keepdims=True)
    ex = jnp.exp(logits - m)
    s = jnp.sum(ex, axis=-1, keepdims=True)
    probs = ex / s
    lse = m + jnp.log(s)
    zpart = jnp.sum(lse * lse)

    iota = jax.lax.broadcasted_iota(jnp.int32, (BT, E), 1)
    m1 = jnp.max(probs, axis=-1, keepdims=True)
    i1 = jnp.min(jnp.where(probs == m1, iota, E), axis=-1, keepdims=True)
    probs_m = jnp.where(iota == i1, -1.0, probs)
    m2 = jnp.max(probs_m, axis=-1, keepdims=True)
    i2 = jnp.min(jnp.where(probs_m == m2, iota, E), axis=-1, keepdims=True)
    denom = m1 + m2 + EPS
    eid_ref[...] = jnp.concatenate([i1, i2], axis=1)
    w_ref[...] = jnp.concatenate([m1 / denom, m2 / denom], axis=1)

    cnt = jnp.sum((iota == i1).astype(jnp.float32)
                  + (iota == i2).astype(jnp.float32), axis=0, keepdims=True)
    cntb_ref[...] = jnp.concatenate(
        [cnt, jnp.zeros((1, 128 - E), jnp.float32)], axis=1)[None]
    psum = jnp.sum(probs, axis=0, keepdims=True)
    ps_safe = jnp.clip(probs, EPS, 1.0)
    entpart = jnp.sum(-ps_safe * jnp.log(ps_safe))

    @pl.when(n == 0)
    def _():
        acc_ref[...] = jnp.zeros_like(acc_ref)

    acc_ref[0:1, 0:E] = acc_ref[0:1, 0:E] + cnt
    acc_ref[1:2, 0:E] = acc_ref[1:2, 0:E] + psum
    acc_ref[2:3, 0:1] = acc_ref[2:3, 0:1] + zpart
    acc_ref[3:4, 0:1] = acc_ref[3:4, 0:1] + entpart

    @pl.when(n == NBR - 1)
    def _():
        cnts = acc_ref[0:1, 0:E]
        ps = acc_ref[1:2, 0:E]
        lb = E * jnp.sum((cnts / R) * (ps / T))
        z = jnp.sum(acc_ref[2:3, 0:1]) / T * 0.001
        ent = (jnp.log(jnp.float32(E)) - jnp.sum(acc_ref[3:4, 0:1]) / T) * 0.01
        aux_ref[...] = jnp.broadcast_to(lb + z + ent, (1, 128))


def _router(x2d, gate_w):
    return pl.pallas_call(
        _router_body,
        grid=(NBR,),
        in_specs=[
            pl.BlockSpec((BT, H), lambda n: (n, 0)),
            pl.BlockSpec((E, H), lambda n: (0, 0)),
        ],
        out_specs=[
            pl.BlockSpec((BT, 2), lambda n: (n, 0)),
            pl.BlockSpec((BT, 2), lambda n: (n, 0)),
            pl.BlockSpec((1, 128), lambda n: (0, 0)),
            pl.BlockSpec((1, 1, 128), lambda n: (n, 0, 0)),
        ],
        out_shape=[
            jax.ShapeDtypeStruct((T, 2), jnp.int32),
            jax.ShapeDtypeStruct((T, 2), jnp.float32),
            jax.ShapeDtypeStruct((1, 128), jnp.float32),
            jax.ShapeDtypeStruct((NBR, 1, 128), jnp.float32),
        ],
        scratch_shapes=[pltpu.VMEM((8, 128), jnp.float32)],
    )(x2d, gate_w)


def _moe_ffn_body(bexp_ref, nused_ref, xs_ref, wg_ref, wu_ref, wd_ref,
                  ws_ref, ys_ref):
    n = pl.program_id(0)

    @pl.when(n < nused_ref[0])
    def _():
        x = xs_ref[...]
        g = jnp.dot(x, wg_ref[0].T, preferred_element_type=jnp.float32)
        u = jnp.dot(x, wu_ref[0].T, preferred_element_type=jnp.float32)
        h = jax.nn.silu(g) * u
        part = jnp.dot(h, wd_ref[0].T, preferred_element_type=jnp.float32)
        ys_ref[...] = part * ws_ref[...]


def _moe_ffn(bexp, nused, xs, egw, euw, edw, ws2d):
    grid_spec = pltpu.PrefetchScalarGridSpec(
        num_scalar_prefetch=2,
        grid=(NBLK,),
        in_specs=[
            pl.BlockSpec((BLK, H), lambda n, bexp, nu: (n, 0)),
            pl.BlockSpec((1, I, H), lambda n, bexp, nu: (bexp[n], 0, 0)),
            pl.BlockSpec((1, I, H), lambda n, bexp, nu: (bexp[n], 0, 0)),
            pl.BlockSpec((1, H, I), lambda n, bexp, nu: (bexp[n], 0, 0)),
            pl.BlockSpec((BLK, 1), lambda n, bexp, nu: (n, 0)),
        ],
        out_specs=pl.BlockSpec((BLK, H), lambda n, bexp, nu: (n, 0)),
    )
    return pl.pallas_call(
        _moe_ffn_body,
        grid_spec=grid_spec,
        out_shape=jax.ShapeDtypeStruct((NPAD, H), jnp.float32),
    )(bexp, nused, xs, egw, euw, edw, ws2d)


def _shared_body(sgp_ref, x_ref, wg_ref, wu_ref, wd_ref, o_ref):
    x = x_ref[...]
    g = jnp.dot(x, wg_ref[...].T, preferred_element_type=jnp.float32)
    u = jnp.dot(x, wu_ref[...].T, preferred_element_type=jnp.float32)
    h = jax.nn.silu(g) * u
    part = jnp.dot(h, wd_ref[...].T, preferred_element_type=jnp.float32)
    scale = jax.nn.sigmoid(sgp_ref[0])
    o_ref[...] = part * scale


def _shared_ffn(x2d, sgw, suw, sdw, sgp):
    grid_spec = pltpu.PrefetchScalarGridSpec(
        num_scalar_prefetch=1,
        grid=(T // BLK,),
        in_specs=[
            pl.BlockSpec((BLK, H), lambda n, sgp: (n, 0)),
            pl.BlockSpec((I, H), lambda n, sgp: (0, 0)),
            pl.BlockSpec((I, H), lambda n, sgp: (0, 0)),
            pl.BlockSpec((H, I), lambda n, sgp: (0, 0)),
        ],
        out_specs=pl.BlockSpec((BLK, H), lambda n, sgp: (n, 0)),
    )
    return pl.pallas_call(
        _shared_body,
        grid_spec=grid_spec,
        out_shape=jax.ShapeDtypeStruct((T, H), jnp.float32),
    )(sgp, x2d, sgw, suw, sdw)


NC, NS, L = 2, 16, 16
NW = NC * NS
TPT = T // NW
RPT = R // NW
NV = R // L
NBLK_PAD = 48

_sc_mesh = plsc.VectorSubcoreMesh(core_axis_name="c", subcore_axis_name="s")


def _dispatch_body(eids_hbm, wts_hbm, x_hbm, cntb_hbm,
                   pos_hbm, ws_hbm, bexp_hbm, nused_hbm, xs_hbm,
                   eids_v, cnt_v, pos_loc, w_loc, idx128a, idx128b,
                   idx0, idx1, xrows,
                   bexp_s, nused_s, sem_a, sem_b, lsems, ssems):
    wid = lax.axis_index("s") * NC + lax.axis_index("c")
    base_r = wid * RPT
    base_t = wid * TPT
    b0 = wid >> 2
    sub = wid & 3
    lane = lax.iota(jnp.int32, L)

    cpe = pltpu.async_copy(eids_hbm.at[pl.ds(b0 * (BT * K), BT * K)],
                           eids_v, sem_a)
    pltpu.sync_copy(cntb_hbm, cnt_v)
    cpe.wait()

    cnt_all = []
    cnt_blkpre = []
    valid = lane < NBR
    before = lane < b0
    for e in range(E):
        g = plsc.load_gather(cnt_v, [jnp.where(valid, lane * 128 + e, 0)])
        cnt_all.append(jnp.sum(jnp.where(valid, g, 0.0)).astype(jnp.int32))
        cnt_blkpre.append(jnp.sum(jnp.where(before, g, 0.0)).astype(jnp.int32))

    def count_body(i, carry):
        v = eids_v[pl.ds(i * L, L)]
        return tuple(carry[e] + jnp.where(v == e, 1, 0) for e in range(E))

    zero = tuple(jnp.zeros((L,), jnp.int32) for _ in range(E))
    acc = lax.fori_loop(0, sub * (RPT // L), count_body, zero)
    cnt_pre = [cnt_blkpre[e] + jnp.sum(acc[e]) for e in range(E)]

    pad = [((cnt_all[e] + (BLK - 1)) >> BLK_SHIFT) << BLK_SHIFT
           for e in range(E)]
    start = []
    run = jnp.int32(0)
    for e in range(E):
        start.append(run)
        run = run + pad[e]
    base = [start[e] + cnt_pre[e] for e in range(E)]

    loc0 = sub * RPT
    running = list(base)
    for j in range(RPT // L):
        v = eids_v[pl.ds(loc0 + j * L, L)]
        posv = jnp.zeros((L,), jnp.int32)
        for e in range(E):
            m = v == e
            mi = jnp.where(m, 1, 0)
            c = jnp.cumsum(mi)
            posv = jnp.where(m, running[e] + c - 1, posv)
            running[e] = running[e] + jnp.sum(mi)
        pos_loc[pl.ds(j * L, L)] = posv

    cpp = pltpu.async_copy(pos_loc, pos_hbm.at[pl.ds(base_r, RPT)], sem_a)

    pltpu.sync_copy(wts_hbm.at[pl.ds(base_r, RPT)], w_loc)
    for j in range(8):
        idx128a[pl.ds(j * L, L)] = pos_loc[pl.ds(j * L, L)]
        idx128b[pl.ds(j * L, L)] = pos_loc[pl.ds(128 + j * L, L)]
    cw0 = pltpu.async_copy(w_loc.at[pl.ds(0, 128)],
                           ws_hbm.at[idx128a], sem_a)
    cw1 = pltpu.async_copy(w_loc.at[pl.ds(128, 128)],
                           ws_hbm.at[idx128b], sem_b)

    CH = 32
    NCHD = TPT // CH
    NJ = CH // L

    def load(c, b):
        pltpu.async_copy(x_hbm.at[pl.ds(base_t + c * CH, CH)],
                         xrows.at[b], lsems.at[b])

    load(0, 0)
    cw0.wait()
    cw1.wait()
    cpp.wait()
    for c in range(NCHD):
        b = c % 2
        pltpu.make_async_copy(x_hbm.at[pl.ds(0, CH)], xrows.at[b],
                              lsems.at[b]).wait()
        for j in range(NJ):
            idxs = c * 2 * CH + j * 2 * L + lane * 2
            idx0[b, pl.ds(j * L, L)] = plsc.load_gather(pos_loc, [idxs])
            idx1[b, pl.ds(j * L, L)] = plsc.load_gather(pos_loc, [idxs + 1])
        pltpu.async_copy(xrows.at[b], xs_hbm.at[idx0.at[b]], ssems.at[b])
        pltpu.async_copy(xrows.at[b], xs_hbm.at[idx1.at[b]], ssems.at[b])
        if c + 1 < NCHD:
            b2 = (c + 1) % 2
            if c >= 1:
                for _ in range(2):
                    pltpu.make_async_copy(xrows.at[b2],
                                          xs_hbm.at[pl.ds(0, CH)],
                                          ssems.at[b2]).wait()
            load(c + 1, b2)
    for b in range(2):
        for _ in range(2):
            pltpu.make_async_copy(xrows.at[b], xs_hbm.at[pl.ds(0, CH)],
                                  ssems.at[b]).wait()

    endblk = [(start[e] + pad[e]) >> BLK_SHIFT for e in range(E)]
    for j in range(NBLK_PAD // L):
        bid = j * L + lane
        sacc = jnp.zeros((L,), jnp.int32)
        for e in range(E):
            sacc = sacc + jnp.where(bid >= endblk[e], 1, 0)
        bexp_s[pl.ds(j * L, L)] = jnp.minimum(sacc, E - 1)
    nused_s[...] = jnp.zeros((L,), jnp.int32) + endblk[E - 1]

    @pl.when(wid == 0)
    def _():
        pltpu.sync_copy(bexp_s, bexp_hbm)
        pltpu.sync_copy(nused_s, nused_hbm)


def _dispatch(eids_flat, wts_flat, x2d, cntb_flat):
    return pl.kernel(
        _dispatch_body,
        out_type=[
            jax.ShapeDtypeStruct((R,), jnp.int32),
            jax.ShapeDtypeStruct((NPAD,), jnp.float32),
            jax.ShapeDtypeStruct((NBLK_PAD,), jnp.int32),
            jax.ShapeDtypeStruct((L,), jnp.int32),
            jax.ShapeDtypeStruct((NPAD, H), jnp.float32),
        ],
        mesh=_sc_mesh,
        compiler_params=pltpu.CompilerParams(needs_layout_passes=False),
        scratch_types=[
            pltpu.VMEM((BT * K,), jnp.int32),
            pltpu.VMEM((NBR * 128,), jnp.float32),
            pltpu.VMEM((RPT,), jnp.int32),
            pltpu.VMEM((RPT,), jnp.float32),
            pltpu.VMEM((128,), jnp.int32),
            pltpu.VMEM((128,), jnp.int32),
            pltpu.VMEM((2, 32), jnp.int32),
            pltpu.VMEM((2, 32), jnp.int32),
            pltpu.VMEM((2, 32, H), jnp.float32),
            pltpu.VMEM((NBLK_PAD,), jnp.int32),
            pltpu.VMEM((L,), jnp.int32),
            pltpu.SemaphoreType.DMA,
            pltpu.SemaphoreType.DMA,
            pltpu.SemaphoreType.DMA((2,)),
            pltpu.SemaphoreType.DMA((2,)),
        ],
    )(eids_flat, wts_flat, x2d, cntb_flat)


NCH = TPT // L


def _combine_body(ys_hbm, ysh_hbm, pos_hbm, out_hbm,
                  pos_v, idx0, idx1, r0, r1, sh, ov, gsems):
    wid = lax.axis_index("s") * NC + lax.axis_index("c")
    base_t = wid * TPT
    base_r = wid * RPT
    pltpu.sync_copy(pos_hbm.at[pl.ds(base_r, RPT)], pos_v)
    lane = lax.iota(jnp.int32, L)

    def issue(c, b):
        t0 = base_t + c * L
        idxs = c * 32 + lane * 2
        idx0[b, pl.ds(0, L)] = plsc.load_gather(pos_v, [idxs])
        idx1[b, pl.ds(0, L)] = plsc.load_gather(pos_v, [idxs + 1])
        pltpu.async_copy(ys_hbm.at[idx0.at[b]], r0.at[b], gsems.at[b])
        pltpu.async_copy(ys_hbm.at[idx1.at[b]], r1.at[b], gsems.at[b])
        pltpu.async_copy(ysh_hbm.at[pl.ds(t0, L)], sh.at[b], gsems.at[b])

    issue(0, 0)
    for c in range(NCH):
        b = c % 2
        if c + 1 < NCH:
            issue(c + 1, (c + 1) % 2)
        pltpu.make_async_copy(ys_hbm.at[pl.ds(0, L)], r0.at[b],
                              gsems.at[b]).wait()
        pltpu.make_async_copy(ys_hbm.at[pl.ds(0, L)], r1.at[b],
                              gsems.at[b]).wait()
        pltpu.make_async_copy(ysh_hbm.at[pl.ds(0, L)], sh.at[b],
                              gsems.at[b]).wait()
        def add_body(col, _):
            o = col * L
            for t in range(L):
                ov[t, pl.ds(o, L)] = (r0[b, t, pl.ds(o, L)]
                                      + r1[b, t, pl.ds(o, L)]
                                      + sh[b, t, pl.ds(o, L)])
            return 0

        lax.fori_loop(0, H // L, add_body, 0)
        pltpu.sync_copy(ov, out_hbm.at[pl.ds(base_t + c * L, L)])


def _combine(ys, ysh, pos):
    return pl.kernel(
        _combine_body,
        out_type=jax.ShapeDtypeStruct((T, H), jnp.float32),
        mesh=_sc_mesh,
        compiler_params=pltpu.CompilerParams(needs_layout_passes=False),
        scratch_types=[
            pltpu.VMEM((RPT,), jnp.int32),
            pltpu.VMEM((2, L), jnp.int32),
            pltpu.VMEM((2, L), jnp.int32),
            pltpu.VMEM((2, L, H), jnp.float32),
            pltpu.VMEM((2, L, H), jnp.float32),
            pltpu.VMEM((2, L, H), jnp.float32),
            pltpu.VMEM((L, H), jnp.float32),
            pltpu.SemaphoreType.DMA((2,)),
        ],
    )(ys, ysh, pos)


def kernel(hidden_states, gate_w, expert_gate_w, expert_up_w, expert_down_w,
           shared_gate_w, shared_up_w, shared_down_w, shared_gate_param):
    x2d = hidden_states.reshape(T, H)

    eids, wts, aux, cntb = _router(x2d, gate_w)

    pos, ws, bexp, nused, xs = _dispatch(eids.reshape(R), wts.reshape(R),
                                         x2d, cntb.reshape(NBR * 128))

    ys = _moe_ffn(bexp, nused, xs, expert_gate_w, expert_up_w, expert_down_w,
                  ws.reshape(NPAD, 1))
    ysh = _shared_ffn(x2d, shared_gate_w, shared_up_w, shared_down_w,
                      shared_gate_param)

    out = _combine(ys, ysh, pos)

    return out.reshape(B, S, H), aux[0, 0]
